# baseline (speedup 1.0000x reference)
import functools

import jax
import jax.numpy as jnp
from jax import lax
from jax.experimental import pallas as pl
from jax.experimental.pallas import tpu as pltpu
from jax.experimental.pallas import tpu_sc as plsc

N_NODES = 10000
N_EDGES = 320000
IN_CH = 128
OUT_CH = 16
HEADS = 8
HC = HEADS * OUT_CH

NC = 2
NS = 16
NW = NC * NS
EPT = N_EDGES // NW
K = 40
NCHUNK = EPT // K
NP = 10240
RPT = NP // NS
ZR = 128


def _proj_body(x_ref, w_ref, ma_ref, mb_ref, xp_ref, ta_ref, tb_ref):
    xp = jnp.dot(x_ref[...], w_ref[...], preferred_element_type=jnp.float32)
    xp_ref[...] = xp
    ta_ref[...] = jnp.dot(xp, ma_ref[...], preferred_element_type=jnp.float32)
    tb_ref[...] = jnp.dot(xp, mb_ref[...], preferred_element_type=jnp.float32)


def _project(x, W, Ma, Mb):
    NB = 10
    R = N_NODES // NB
    return pl.pallas_call(
        _proj_body,
        grid=(NB,),
        in_specs=[
            pl.BlockSpec((R, IN_CH), lambda i: (i, 0)),
            pl.BlockSpec((IN_CH, HC), lambda i: (0, 0)),
            pl.BlockSpec((IN_CH, 2 * HEADS), lambda i: (0, 0)),
            pl.BlockSpec((IN_CH, 2 * HEADS), lambda i: (0, 0)),
        ],
        out_specs=[
            pl.BlockSpec((R, HC), lambda i: (i, 0)),
            pl.BlockSpec((R, 2 * HEADS), lambda i: (i, 0)),
            pl.BlockSpec((R, 2 * HEADS), lambda i: (i, 0)),
        ],
        out_shape=[
            jax.ShapeDtypeStruct((N_NODES, HC), jnp.float32),
            jax.ShapeDtypeStruct((N_NODES, 2 * HEADS), jnp.float32),
            jax.ShapeDtypeStruct((N_NODES, 2 * HEADS), jnp.float32),
        ],
    )(x, W, Ma, Mb)


NRING = 4


def _edge_body(row_hbm, col_hbm, ta_hbm, tb_hbm, xp_hbm,
               raw_out, den_out,
               ri, ci, tr, tc, w, xp,
               raw_acc, den_acc,
               isem, gsem, ssem):
    c = lax.axis_index("c")
    s = lax.axis_index("s")
    ebase = (c * NS + s) * EPT

    def _zrow(i, carry):
        for h in range(HEADS):
            xp[0][i, pl.ds(h * 16, 16)] = jnp.zeros((16,), jnp.float32)
        w[0][i, :] = jnp.zeros((16,), jnp.float32)
        return carry

    lax.fori_loop(0, K, _zrow, 0)
    base_r = s * RPT
    for kk in range(RPT // K):
        pltpu.sync_copy(xp[0], raw_acc.at[pl.ds(base_r + kk * K, K)])
        pltpu.sync_copy(w[0], den_acc.at[pl.ds(base_r + kk * K, K)])
    plsc.subcore_barrier()

    def _iload(cc, isl):
        off = pl.multiple_of(ebase + cc * K, 8)
        pltpu.async_copy(row_hbm.at[pl.ds(off, K)], ri[isl], isem[isl])
        pltpu.async_copy(col_hbm.at[pl.ds(off, K)], ci[isl], isem[isl])

    def _iwait(isl):
        pltpu.make_async_copy(row_hbm.at[pl.ds(0, K)], ri[isl], isem[isl]).wait()
        pltpu.make_async_copy(col_hbm.at[pl.ds(0, K)], ci[isl], isem[isl]).wait()

    def _issue(ds, isl):
        pltpu.async_copy(xp_hbm.at[ri[isl]], xp[ds], gsem[ds])
        pltpu.async_copy(ta_hbm.at[ri[isl]], tr[ds], gsem[ds])
        pltpu.async_copy(tb_hbm.at[ci[isl]], tc[ds], gsem[ds])

    def _gwait(ds, isl):
        pltpu.make_async_copy(xp_hbm.at[ri[isl]], xp[ds], gsem[ds]).wait()
        pltpu.make_async_copy(ta_hbm.at[ri[isl]], tr[ds], gsem[ds]).wait()
        pltpu.make_async_copy(tb_hbm.at[ci[isl]], tc[ds], gsem[ds]).wait()

    def _scat(ds, isl):
        pltpu.async_copy(w[ds], den_acc.at[ci[isl]], ssem[ds], add=True)
        pltpu.async_copy(xp[ds], raw_acc.at[ci[isl]], ssem[ds], add=True)

    def _sdrain(ds, isl):
        pltpu.make_async_copy(w[ds], den_acc.at[ci[isl]], ssem[ds]).wait()
        pltpu.make_async_copy(xp[ds], raw_acc.at[ci[isl]], ssem[ds]).wait()

    def _compute(ds):
        trb, tcb, wb, xpb = tr[ds], tc[ds], w[ds], xp[ds]

        @plsc.parallel_loop(0, K, unroll=8)
        def _edge(i):
            w16 = jnp.exp(trb[i, :] + tcb[i, :])
            wb[i, :] = w16
            for h in range(HEADS):
                wh = jnp.broadcast_to(w16[h], (16,))
                xpb[i, pl.ds(h * 16, 16)] = xpb[i, pl.ds(h * 16, 16)] * wh

    NI = 2 * NRING
    for cc in range(4):
        _iload(cc, cc)
    _iwait(0)
    _issue(0, 0)
    _iwait(1)
    _issue(1, 1)

    def _outer(jj, carry):
        for k in range(NI):
            cc = NI * jj + k
            ds = k % NRING
            _iwait((k + 2) % NI)
            _issue((k + 2) % NRING, (k + 2) % NI)
            _gwait(ds, k)
            _compute(ds)
            if k == 0:
                @pl.when(jj >= 1)
                def _():
                    _sdrain((k + 3) % NRING, (k + 7) % NI)
            else:
                _sdrain((k + 3) % NRING, (k + 7) % NI)
            if k >= 6:
                @pl.when(jj < (NCHUNK - 10) // NI)
                def _():
                    _iload(cc + 4, (k + 4) % NI)
            else:
                _iload(cc + 4, (k + 4) % NI)
            _scat(ds, k)
        return carry

    lax.fori_loop(0, (NCHUNK - 2) // NI, _outer, 0)
    _gwait(0, 0)
    _compute(0)
    _sdrain(3, 7)
    _scat(0, 0)
    _gwait(1, 1)
    _compute(1)
    _scat(1, 1)
    _sdrain(0, 0)
    _sdrain(1, 1)
    plsc.subcore_barrier()

    pltpu.sync_copy(raw_acc.at[pl.ds(base_r, RPT)],
                    raw_out.at[c, pl.ds(base_r, RPT)])
    pltpu.sync_copy(den_acc.at[pl.ds(base_r, RPT)],
                    den_out.at[c, pl.ds(base_r, RPT)])


def _edge_body_flat(row_hbm, col_hbm, ta_hbm, tb_hbm, xp_hbm,
                    raw_out, den_out, *rest):
    ri = rest[0:8]
    ci = rest[8:16]
    tr = rest[16:20]
    tc = rest[20:24]
    w = rest[24:28]
    xp = rest[28:32]
    raw_acc, den_acc = rest[32], rest[33]
    isem = rest[34:42]
    gsem = rest[42:46]
    ssem = rest[46:50]
    _edge_body(row_hbm, col_hbm, ta_hbm, tb_hbm, xp_hbm, raw_out, den_out,
               ri, ci, tr, tc, w, xp, raw_acc, den_acc, isem, gsem, ssem)


def _edge_pass(row, col, ta, tb, xp):
    mesh = plsc.VectorSubcoreMesh(core_axis_name="c", subcore_axis_name="s")
    f = functools.partial(
        pl.kernel,
        mesh=mesh,
        compiler_params=pltpu.CompilerParams(use_tc_tiling_on_sc=False),
        out_type=[
            jax.ShapeDtypeStruct((NC, NP, HC), jnp.float32),
            jax.ShapeDtypeStruct((NC, NP, 2 * HEADS), jnp.float32),
        ],
        scratch_types=[
            *[pltpu.VMEM((K,), jnp.int32) for _ in range(2 * NRING)],
            *[pltpu.VMEM((K,), jnp.int32) for _ in range(2 * NRING)],
            *[pltpu.VMEM((K, 2 * HEADS), jnp.float32)
              for _ in range(3 * NRING)],
            *[pltpu.VMEM((K, HC), jnp.float32) for _ in range(NRING)],
            pltpu.VMEM_SHARED((NP, HC), jnp.float32),
            pltpu.VMEM_SHARED((NP, 2 * HEADS), jnp.float32),
            *[pltpu.SemaphoreType.DMA for _ in range(4 * NRING)],
        ],
    )(_edge_body_flat)
    return f(row, col, ta, tb, xp)


def _norm_body(raw_ref, den_ref, r_ref, o_ref):
    r = raw_ref[0] + raw_ref[1]
    d = den_ref[0] + den_ref[1]
    dx = jnp.dot(d, r_ref[...], preferred_element_type=jnp.float32)
    o_ref[...] = r / (dx + 1e-16)


def _normalize(raw, den, Rmat):
    NB = 10
    R = N_NODES // NB
    return pl.pallas_call(
        _norm_body,
        grid=(NB,),
        in_specs=[
            pl.BlockSpec((NC, R, HC), lambda i: (0, i, 0)),
            pl.BlockSpec((NC, R, 2 * HEADS), lambda i: (0, i, 0)),
            pl.BlockSpec((2 * HEADS, HC), lambda i: (0, 0)),
        ],
        out_specs=pl.BlockSpec((R, HC), lambda i: (i, 0)),
        out_shape=jax.ShapeDtypeStruct((N_NODES, HC), jnp.float32),
    )(raw, den, Rmat)


def _block_diag(a):
    cc = jnp.arange(HC)
    sel = (cc[:, None] // OUT_CH) == jnp.arange(HEADS)[None, :]
    vals = a.reshape(HC)[cc]
    return sel.astype(jnp.float32) * vals[:, None]


def kernel(x, edge_index, W, a_src, a_dst):
    ei = edge_index.astype(jnp.int32)
    row, col = ei[0], ei[1]
    Ma = jnp.concatenate([_block_diag(a_src), _block_diag(a_dst)], axis=1)
    Mb = jnp.concatenate([_block_diag(a_dst), _block_diag(a_src)], axis=1)
    jj = jnp.arange(HC)[None, :]
    hh = jnp.arange(2 * HEADS)[:, None]
    Rmat = ((jj // OUT_CH == hh) & (hh < HEADS)).astype(jnp.float32)

    xp, ta, tb = _project(x, W, Ma, Mb)
    raw, den = _edge_pass(row, col, ta, tb, xp)
    return _normalize(raw, den, Rmat)


# speedup vs baseline: 1.3714x; 1.3714x over previous
import functools

import jax
import jax.numpy as jnp
from jax import lax
from jax.experimental import pallas as pl
from jax.experimental.pallas import tpu as pltpu
from jax.experimental.pallas import tpu_sc as plsc

N_NODES = 10000
N_EDGES = 320000
IN_CH = 128
OUT_CH = 16
HEADS = 8
HC = HEADS * OUT_CH
TW = HC + 2 * HEADS

NC = 2
NS = 16
NW = NC * NS
EPT = N_EDGES // NW
K = 40
NCHUNK = EPT // K
NP = 10240
RPT = NP // NS
NRING = 4


def _proj_body(x_ref, w_ref, ma_ref, mb_ref, xcat_ref, tb_ref):
    xp = jnp.dot(x_ref[...], w_ref[...], preferred_element_type=jnp.float32)
    xcat_ref[:, pl.ds(0, HC)] = xp
    xcat_ref[:, pl.ds(HC, 2 * HEADS)] = jnp.dot(
        xp, ma_ref[...], preferred_element_type=jnp.float32)
    tb_ref[...] = jnp.dot(xp, mb_ref[...], preferred_element_type=jnp.float32)


def _project(x, W, Ma, Mb):
    NB = 10
    R = N_NODES // NB
    return pl.pallas_call(
        _proj_body,
        grid=(NB,),
        in_specs=[
            pl.BlockSpec((R, IN_CH), lambda i: (i, 0)),
            pl.BlockSpec((IN_CH, HC), lambda i: (0, 0)),
            pl.BlockSpec((IN_CH, 2 * HEADS), lambda i: (0, 0)),
            pl.BlockSpec((IN_CH, 2 * HEADS), lambda i: (0, 0)),
        ],
        out_specs=[
            pl.BlockSpec((R, TW), lambda i: (i, 0)),
            pl.BlockSpec((R, 2 * HEADS), lambda i: (i, 0)),
        ],
        out_shape=[
            jax.ShapeDtypeStruct((N_NODES, TW), jnp.float32),
            jax.ShapeDtypeStruct((N_NODES, 2 * HEADS), jnp.float32),
        ],
    )(x, W, Ma, Mb)


def _edge_body(row_hbm, col_hbm, xcat_hbm, tb_hbm,
               acc_out,
               ri, ci, cb, tc, acc,
               isem, gsem, ssem):
    c = lax.axis_index("c")
    s = lax.axis_index("s")
    ebase = (c * NS + s) * EPT

    def _zrow(i, carry):
        for h in range(TW // 16):
            cb[0][i, pl.ds(h * 16, 16)] = jnp.zeros((16,), jnp.float32)
        return carry

    lax.fori_loop(0, K, _zrow, 0)
    base_r = s * RPT
    for kk in range(RPT // K):
        pltpu.sync_copy(cb[0], acc.at[pl.ds(base_r + kk * K, K)])
    plsc.subcore_barrier()

    def _iload(cc, isl):
        off = pl.multiple_of(ebase + cc * K, 8)
        pltpu.async_copy(row_hbm.at[pl.ds(off, K)], ri[isl], isem[isl])
        pltpu.async_copy(col_hbm.at[pl.ds(off, K)], ci[isl], isem[isl])

    def _iwait(isl):
        pltpu.make_async_copy(row_hbm.at[pl.ds(0, K)], ri[isl], isem[isl]).wait()
        pltpu.make_async_copy(col_hbm.at[pl.ds(0, K)], ci[isl], isem[isl]).wait()

    def _issue(ds, isl):
        pltpu.async_copy(xcat_hbm.at[ri[isl]], cb[ds], gsem[ds])
        pltpu.async_copy(tb_hbm.at[ci[isl]], tc[ds], gsem[ds])

    def _gwait(ds, isl):
        pltpu.make_async_copy(xcat_hbm.at[ri[isl]], cb[ds], gsem[ds]).wait()
        pltpu.make_async_copy(tb_hbm.at[ci[isl]], tc[ds], gsem[ds]).wait()

    def _scat(ds, isl):
        pltpu.async_copy(cb[ds], acc.at[ci[isl]], ssem[ds], add=True)

    def _sdrain(ds, isl):
        pltpu.make_async_copy(cb[ds], acc.at[ci[isl]], ssem[ds]).wait()

    def _compute(ds):
        cbb, tcb = cb[ds], tc[ds]

        @plsc.parallel_loop(0, K, unroll=4)
        def _edge(i):
            w16 = jnp.exp(cbb[i, pl.ds(HC, 16)] + tcb[i, :])
            cbb[i, pl.ds(HC, 16)] = w16
            for h in range(HEADS):
                wh = jnp.broadcast_to(w16[h], (16,))
                cbb[i, pl.ds(h * 16, 16)] = cbb[i, pl.ds(h * 16, 16)] * wh

    NI = 2 * NRING
    for cc in range(4):
        _iload(cc, cc)
    _iwait(0)
    _issue(0, 0)
    _iwait(1)
    _issue(1, 1)

    def _outer(jj, carry):
        for k in range(NI):
            cc = NI * jj + k
            ds = k % NRING
            _iwait((k + 2) % NI)
            _issue((k + 2) % NRING, (k + 2) % NI)
            _gwait(ds, k)
            _compute(ds)
            if k == 0:
                @pl.when(jj >= 1)
                def _():
                    _sdrain((k + 3) % NRING, (k + 7) % NI)
            else:
                _sdrain((k + 3) % NRING, (k + 7) % NI)
            if k >= 6:
                @pl.when(jj < (NCHUNK - 10) // NI)
                def _():
                    _iload(cc + 4, (k + 4) % NI)
            else:
                _iload(cc + 4, (k + 4) % NI)
            _scat(ds, k)
        return carry

    lax.fori_loop(0, (NCHUNK - 2) // NI, _outer, 0)
    _gwait(0, 0)
    _compute(0)
    _sdrain(3, 7)
    _scat(0, 0)
    _gwait(1, 1)
    _compute(1)
    _scat(1, 1)
    _sdrain(0, 0)
    _sdrain(1, 1)
    plsc.subcore_barrier()

    pltpu.sync_copy(acc.at[pl.ds(base_r, RPT)],
                    acc_out.at[c, pl.ds(base_r, RPT)])


def _edge_body_flat(row_hbm, col_hbm, xcat_hbm, tb_hbm, acc_out, *rest):
    ri = rest[0:8]
    ci = rest[8:16]
    cb = rest[16:20]
    tc = rest[20:24]
    acc = rest[24]
    isem = rest[25:33]
    gsem = rest[33:37]
    ssem = rest[37:41]
    _edge_body(row_hbm, col_hbm, xcat_hbm, tb_hbm, acc_out,
               ri, ci, cb, tc, acc, isem, gsem, ssem)


def _edge_pass(row, col, xcat, tb):
    mesh = plsc.VectorSubcoreMesh(core_axis_name="c", subcore_axis_name="s")
    f = functools.partial(
        pl.kernel,
        mesh=mesh,
        compiler_params=pltpu.CompilerParams(use_tc_tiling_on_sc=False),
        out_type=jax.ShapeDtypeStruct((NC, NP, TW), jnp.float32),
        scratch_types=[
            *[pltpu.VMEM((K,), jnp.int32) for _ in range(2 * NRING)],
            *[pltpu.VMEM((K,), jnp.int32) for _ in range(2 * NRING)],
            *[pltpu.VMEM((K, TW), jnp.float32) for _ in range(NRING)],
            *[pltpu.VMEM((K, 2 * HEADS), jnp.float32)
              for _ in range(NRING)],
            pltpu.VMEM_SHARED((NP, TW), jnp.float32),
            *[pltpu.SemaphoreType.DMA for _ in range(4 * NRING)],
        ],
    )(_edge_body_flat)
    return f(row, col, xcat, tb)


def _norm_body(acc_ref, r_ref, o_ref):
    r = acc_ref[0, :, pl.ds(0, HC)] + acc_ref[1, :, pl.ds(0, HC)]
    d = acc_ref[0, :, pl.ds(HC, 2 * HEADS)] + acc_ref[1, :, pl.ds(HC, 2 * HEADS)]
    dx = jnp.dot(d, r_ref[...], preferred_element_type=jnp.float32)
    o_ref[...] = r / (dx + 1e-16)


def _normalize(acc, Rmat):
    NB = 10
    R = N_NODES // NB
    return pl.pallas_call(
        _norm_body,
        grid=(NB,),
        in_specs=[
            pl.BlockSpec((NC, R, TW), lambda i: (0, i, 0)),
            pl.BlockSpec((2 * HEADS, HC), lambda i: (0, 0)),
        ],
        out_specs=pl.BlockSpec((R, HC), lambda i: (i, 0)),
        out_shape=jax.ShapeDtypeStruct((N_NODES, HC), jnp.float32),
    )(acc, Rmat)


def _block_diag(a):
    cc = jnp.arange(HC)
    sel = (cc[:, None] // OUT_CH) == jnp.arange(HEADS)[None, :]
    vals = a.reshape(HC)[cc]
    return sel.astype(jnp.float32) * vals[:, None]


def kernel(x, edge_index, W, a_src, a_dst):
    ei = edge_index.astype(jnp.int32)
    row, col = ei[0], ei[1]
    Ma = jnp.concatenate([_block_diag(a_src), _block_diag(a_dst)], axis=1)
    Mb = jnp.concatenate([_block_diag(a_dst), _block_diag(a_src)], axis=1)
    jj = jnp.arange(HC)[None, :]
    hh = jnp.arange(2 * HEADS)[:, None]
    Rmat = ((jj // OUT_CH == hh) & (hh < HEADS)).astype(jnp.float32)

    xcat, tb = _project(x, W, Ma, Mb)
    acc = _edge_pass(row, col, xcat, tb)
    return _normalize(acc, Rmat)


# speedup vs baseline: 1.4426x; 1.0519x over previous
import functools

import jax
import jax.numpy as jnp
from jax import lax
from jax.experimental import pallas as pl
from jax.experimental.pallas import tpu as pltpu
from jax.experimental.pallas import tpu_sc as plsc

N_NODES = 10000
N_EDGES = 320000
IN_CH = 128
OUT_CH = 16
HEADS = 8
HC = HEADS * OUT_CH

NC = 2
NS = 16
NW = NC * NS
EPT = N_EDGES // NW
K = 40
NCHUNK = EPT // K
NP = 10240
RPT = NP // NS
NRING = 4


def _proj_body(x_ref, w_ref, ma_ref, mb_ref, xp_ref, ta_ref, tb_ref):
    xp = jnp.dot(x_ref[...], w_ref[...], preferred_element_type=jnp.float32)
    xp_ref[...] = xp
    ta_ref[...] = jnp.dot(xp, ma_ref[...], preferred_element_type=jnp.float32)
    tb_ref[...] = jnp.dot(xp, mb_ref[...], preferred_element_type=jnp.float32)


def _project(x, W, Ma, Mb):
    NB = 10
    R = N_NODES // NB
    return pl.pallas_call(
        _proj_body,
        grid=(NB,),
        in_specs=[
            pl.BlockSpec((R, IN_CH), lambda i: (i, 0)),
            pl.BlockSpec((IN_CH, HC), lambda i: (0, 0)),
            pl.BlockSpec((IN_CH, 2 * HEADS), lambda i: (0, 0)),
            pl.BlockSpec((IN_CH, 2 * HEADS), lambda i: (0, 0)),
        ],
        out_specs=[
            pl.BlockSpec((R, HC), lambda i: (i, 0)),
            pl.BlockSpec((R, 2 * HEADS), lambda i: (i, 0)),
            pl.BlockSpec((R, 2 * HEADS), lambda i: (i, 0)),
        ],
        out_shape=[
            jax.ShapeDtypeStruct((N_NODES, HC), jnp.float32),
            jax.ShapeDtypeStruct((N_NODES, 2 * HEADS), jnp.float32),
            jax.ShapeDtypeStruct((N_NODES, 2 * HEADS), jnp.float32),
        ],
    )(x, W, Ma, Mb)


def _edge_body(row_hbm, col_hbm, ta_hbm, tb_hbm, xp_hbm,
               raw_out, den_out,
               ri, ci, tr, tc, w, xp,
               raw_acc, den_acc,
               isem, gsem, ssem):
    c = lax.axis_index("c")
    s = lax.axis_index("s")
    ebase = (c * NS + s) * EPT

    def _zrow(i, carry):
        for h in range(HEADS):
            xp[0][i, pl.ds(h * 16, 16)] = jnp.zeros((16,), jnp.float32)
        w[0][i, :] = jnp.zeros((16,), jnp.float32)
        return carry

    lax.fori_loop(0, K, _zrow, 0)
    base_r = s * RPT
    for kk in range(RPT // K):
        pltpu.sync_copy(xp[0], raw_acc.at[pl.ds(base_r + kk * K, K)])
        pltpu.sync_copy(w[0], den_acc.at[pl.ds(base_r + kk * K, K)])
    plsc.subcore_barrier()

    def _iload(cc, isl):
        off = pl.multiple_of(ebase + cc * K, 8)
        pltpu.async_copy(row_hbm.at[pl.ds(off, K)], ri[isl], isem[isl])
        pltpu.async_copy(col_hbm.at[pl.ds(off, K)], ci[isl], isem[isl])

    def _iwait(isl):
        pltpu.make_async_copy(row_hbm.at[pl.ds(0, K)], ri[isl], isem[isl]).wait()
        pltpu.make_async_copy(col_hbm.at[pl.ds(0, K)], ci[isl], isem[isl]).wait()

    def _issue(ds, isl):
        pltpu.async_copy(xp_hbm.at[ri[isl]], xp[ds], gsem[ds])
        pltpu.async_copy(ta_hbm.at[ri[isl]], tr[ds], gsem[ds])
        pltpu.async_copy(tb_hbm.at[ci[isl]], tc[ds], gsem[ds])

    def _gwait(ds, isl):
        pltpu.make_async_copy(xp_hbm.at[ri[isl]], xp[ds], gsem[ds]).wait()
        pltpu.make_async_copy(ta_hbm.at[ri[isl]], tr[ds], gsem[ds]).wait()
        pltpu.make_async_copy(tb_hbm.at[ci[isl]], tc[ds], gsem[ds]).wait()

    def _scat(ds, isl):
        pltpu.async_copy(w[ds], den_acc.at[ci[isl]], ssem[ds], add=True)
        pltpu.async_copy(xp[ds], raw_acc.at[ci[isl]], ssem[ds], add=True)

    def _sdrain(ds, isl):
        pltpu.make_async_copy(w[ds], den_acc.at[ci[isl]], ssem[ds]).wait()
        pltpu.make_async_copy(xp[ds], raw_acc.at[ci[isl]], ssem[ds]).wait()

    def _compute(ds):
        trb, tcb, wb, xpb = tr[ds], tc[ds], w[ds], xp[ds]

        @plsc.parallel_loop(0, K, unroll=4)
        def _edge(i):
            w16 = jnp.exp(trb[i, :] + tcb[i, :])
            wb[i, :] = w16
            for h in range(HEADS):
                wh = jnp.broadcast_to(w16[h], (16,))
                xpb[i, pl.ds(h * 16, 16)] = xpb[i, pl.ds(h * 16, 16)] * wh

    NI = 2 * NRING
    for cc in range(4):
        _iload(cc, cc)
    _iwait(0)
    _issue(0, 0)
    _iwait(1)
    _issue(1, 1)

    def _outer(jj, carry):
        for k in range(NI):
            cc = NI * jj + k
            ds = k % NRING
            _iwait((k + 2) % NI)
            _issue((k + 2) % NRING, (k + 2) % NI)
            _gwait(ds, k)
            _compute(ds)
            if k == 0:
                @pl.when(jj >= 1)
                def _():
                    _sdrain((k + 3) % NRING, (k + 7) % NI)
            else:
                _sdrain((k + 3) % NRING, (k + 7) % NI)
            if k >= 6:
                @pl.when(jj < (NCHUNK - 10) // NI)
                def _():
                    _iload(cc + 4, (k + 4) % NI)
            else:
                _iload(cc + 4, (k + 4) % NI)
            _scat(ds, k)
        return carry

    lax.fori_loop(0, (NCHUNK - 2) // NI, _outer, 0)
    _gwait(0, 0)
    _compute(0)
    _sdrain(3, 7)
    _scat(0, 0)
    _gwait(1, 1)
    _compute(1)
    _scat(1, 1)
    _sdrain(0, 0)
    _sdrain(1, 1)
    plsc.subcore_barrier()

    pltpu.sync_copy(raw_acc.at[pl.ds(base_r, RPT)],
                    raw_out.at[c, pl.ds(base_r, RPT)])
    pltpu.sync_copy(den_acc.at[pl.ds(base_r, RPT)],
                    den_out.at[c, pl.ds(base_r, RPT)])


def _edge_body_flat(row_hbm, col_hbm, ta_hbm, tb_hbm, xp_hbm,
                    raw_out, den_out, *rest):
    ri = rest[0:8]
    ci = rest[8:16]
    tr = rest[16:20]
    tc = rest[20:24]
    w = rest[24:28]
    xp = rest[28:32]
    raw_acc, den_acc = rest[32], rest[33]
    isem = rest[34:42]
    gsem = rest[42:46]
    ssem = rest[46:50]
    _edge_body(row_hbm, col_hbm, ta_hbm, tb_hbm, xp_hbm, raw_out, den_out,
               ri, ci, tr, tc, w, xp, raw_acc, den_acc, isem, gsem, ssem)


def _edge_pass(row, col, ta, tb, xp):
    mesh = plsc.VectorSubcoreMesh(core_axis_name="c", subcore_axis_name="s")
    f = functools.partial(
        pl.kernel,
        mesh=mesh,
        compiler_params=pltpu.CompilerParams(use_tc_tiling_on_sc=False),
        out_type=[
            jax.ShapeDtypeStruct((NC, NP, HC), jnp.float32),
            jax.ShapeDtypeStruct((NC, NP, 2 * HEADS), jnp.float32),
        ],
        scratch_types=[
            *[pltpu.VMEM((K,), jnp.int32) for _ in range(2 * NRING)],
            *[pltpu.VMEM((K,), jnp.int32) for _ in range(2 * NRING)],
            *[pltpu.VMEM((K, 2 * HEADS), jnp.float32)
              for _ in range(3 * NRING)],
            *[pltpu.VMEM((K, HC), jnp.float32) for _ in range(NRING)],
            pltpu.VMEM_SHARED((NP, HC), jnp.float32),
            pltpu.VMEM_SHARED((NP, 2 * HEADS), jnp.float32),
            *[pltpu.SemaphoreType.DMA for _ in range(4 * NRING)],
        ],
    )(_edge_body_flat)
    return f(row, col, ta, tb, xp)


def _norm_body(raw_ref, den_ref, r_ref, o_ref):
    r = raw_ref[0] + raw_ref[1]
    d = den_ref[0] + den_ref[1]
    dx = jnp.dot(d, r_ref[...], preferred_element_type=jnp.float32)
    o_ref[...] = r / (dx + 1e-16)


def _normalize(raw, den, Rmat):
    NB = 10
    R = N_NODES // NB
    return pl.pallas_call(
        _norm_body,
        grid=(NB,),
        in_specs=[
            pl.BlockSpec((NC, R, HC), lambda i: (0, i, 0)),
            pl.BlockSpec((NC, R, 2 * HEADS), lambda i: (0, i, 0)),
            pl.BlockSpec((2 * HEADS, HC), lambda i: (0, 0)),
        ],
        out_specs=pl.BlockSpec((R, HC), lambda i: (i, 0)),
        out_shape=jax.ShapeDtypeStruct((N_NODES, HC), jnp.float32),
    )(raw, den, Rmat)


def _block_diag(a):
    cc = jnp.arange(HC)
    sel = (cc[:, None] // OUT_CH) == jnp.arange(HEADS)[None, :]
    vals = a.reshape(HC)[cc]
    return sel.astype(jnp.float32) * vals[:, None]


def kernel(x, edge_index, W, a_src, a_dst):
    ei = edge_index.astype(jnp.int32)
    row, col = ei[0], ei[1]
    Ma = jnp.concatenate([_block_diag(a_src), _block_diag(a_dst)], axis=1)
    Mb = jnp.concatenate([_block_diag(a_dst), _block_diag(a_src)], axis=1)
    jj = jnp.arange(HC)[None, :]
    hh = jnp.arange(2 * HEADS)[:, None]
    Rmat = ((jj // OUT_CH == hh) & (hh < HEADS)).astype(jnp.float32)

    xp, ta, tb = _project(x, W, Ma, Mb)
    raw, den = _edge_pass(row, col, ta, tb, xp)
    return _normalize(raw, den, Rmat)


# speedup vs baseline: 1.4883x; 1.0317x over previous
import functools

import jax
import jax.numpy as jnp
from jax import lax
from jax.experimental import pallas as pl
from jax.experimental.pallas import tpu as pltpu
from jax.experimental.pallas import tpu_sc as plsc

N_NODES = 10000
N_EDGES = 320000
IN_CH = 128
OUT_CH = 16
HEADS = 8
HC = HEADS * OUT_CH

NC = 2
NS = 16
NW = NC * NS
EPT = N_EDGES // NW
K = 40
NCHUNK = EPT // K
NP = 10240
RPT = NP // NS
NRING = 4


def _proj_body(x_ref, w_ref, ma_ref, mb_ref, xp_ref, ta_ref, tb_ref):
    xp = jnp.dot(x_ref[...], w_ref[...], preferred_element_type=jnp.float32)
    xp_ref[...] = xp
    ta_ref[...] = jnp.dot(xp, ma_ref[...], preferred_element_type=jnp.float32)
    tb_ref[...] = jnp.dot(xp, mb_ref[...], preferred_element_type=jnp.float32)


def _project(x, W, Ma, Mb):
    NB = 10
    R = N_NODES // NB
    return pl.pallas_call(
        _proj_body,
        grid=(NB,),
        in_specs=[
            pl.BlockSpec((R, IN_CH), lambda i: (i, 0)),
            pl.BlockSpec((IN_CH, HC), lambda i: (0, 0)),
            pl.BlockSpec((IN_CH, 2 * HEADS), lambda i: (0, 0)),
            pl.BlockSpec((IN_CH, 2 * HEADS), lambda i: (0, 0)),
        ],
        out_specs=[
            pl.BlockSpec((R, HC), lambda i: (i, 0)),
            pl.BlockSpec((R, 2 * HEADS), lambda i: (i, 0)),
            pl.BlockSpec((R, 2 * HEADS), lambda i: (i, 0)),
        ],
        out_shape=[
            jax.ShapeDtypeStruct((N_NODES, HC), jnp.float32),
            jax.ShapeDtypeStruct((N_NODES, 2 * HEADS), jnp.float32),
            jax.ShapeDtypeStruct((N_NODES, 2 * HEADS), jnp.float32),
        ],
    )(x, W, Ma, Mb)


def _edge_body(row_hbm, col_hbm, ta_hbm, tb_hbm, xp_hbm,
               raw_out, den_out,
               ri, ci, tr, tc, w, xp,
               raw_acc, den_acc,
               isem, gsem, ssem):
    c = lax.axis_index("c")
    s = lax.axis_index("s")
    ebase = (c * NS + s) * EPT

    def _zrow(i, carry):
        for h in range(HEADS):
            xp[0][i, pl.ds(h * 16, 16)] = jnp.zeros((16,), jnp.float32)
        w[0][i, :] = jnp.zeros((16,), jnp.float32)
        return carry

    lax.fori_loop(0, K, _zrow, 0)
    base_r = s * RPT
    for kk in range(RPT // K):
        pltpu.async_copy(xp[0], raw_acc.at[pl.ds(base_r + kk * K, K)],
                         gsem[kk % NRING])
        pltpu.async_copy(w[0], den_acc.at[pl.ds(base_r + kk * K, K)],
                         ssem[kk % NRING])
    for kk in range(RPT // K):
        pltpu.make_async_copy(xp[0], raw_acc.at[pl.ds(base_r + kk * K, K)],
                              gsem[kk % NRING]).wait()
        pltpu.make_async_copy(w[0], den_acc.at[pl.ds(base_r + kk * K, K)],
                              ssem[kk % NRING]).wait()
    plsc.subcore_barrier()

    def _iload(cc, isl):
        off = pl.multiple_of(ebase + cc * K, 8)
        pltpu.async_copy(row_hbm.at[pl.ds(off, K)], ri[isl], isem[isl])
        pltpu.async_copy(col_hbm.at[pl.ds(off, K)], ci[isl], isem[isl])

    def _iwait(isl):
        pltpu.make_async_copy(row_hbm.at[pl.ds(0, K)], ri[isl], isem[isl]).wait()
        pltpu.make_async_copy(col_hbm.at[pl.ds(0, K)], ci[isl], isem[isl]).wait()

    def _issue(ds, isl):
        pltpu.async_copy(xp_hbm.at[ri[isl]], xp[ds], gsem[ds])
        pltpu.async_copy(ta_hbm.at[ri[isl]], tr[ds], gsem[ds])
        pltpu.async_copy(tb_hbm.at[ci[isl]], tc[ds], gsem[ds])

    def _gwait(ds, isl):
        pltpu.make_async_copy(xp_hbm.at[ri[isl]], xp[ds], gsem[ds]).wait()
        pltpu.make_async_copy(ta_hbm.at[ri[isl]], tr[ds], gsem[ds]).wait()
        pltpu.make_async_copy(tb_hbm.at[ci[isl]], tc[ds], gsem[ds]).wait()

    def _scat(ds, isl):
        pltpu.async_copy(w[ds], den_acc.at[ci[isl]], ssem[ds], add=True)
        pltpu.async_copy(xp[ds], raw_acc.at[ci[isl]], ssem[ds], add=True)

    def _sdrain(ds, isl):
        pltpu.make_async_copy(w[ds], den_acc.at[ci[isl]], ssem[ds]).wait()
        pltpu.make_async_copy(xp[ds], raw_acc.at[ci[isl]], ssem[ds]).wait()

    def _compute(ds):
        trb, tcb, wb, xpb = tr[ds], tc[ds], w[ds], xp[ds]

        @plsc.parallel_loop(0, K, unroll=4)
        def _edge(i):
            w16 = jnp.exp(trb[i, :] + tcb[i, :])
            wb[i, :] = w16
            for h in range(HEADS):
                wh = jnp.broadcast_to(w16[h], (16,))
                xpb[i, pl.ds(h * 16, 16)] = xpb[i, pl.ds(h * 16, 16)] * wh

    NI = 2 * NRING
    for cc in range(4):
        _iload(cc, cc)
    _iwait(0)
    _issue(0, 0)
    _iwait(1)
    _issue(1, 1)

    def _outer(jj, carry):
        for k in range(NI):
            cc = NI * jj + k
            ds = k % NRING
            _iwait((k + 2) % NI)
            _issue((k + 2) % NRING, (k + 2) % NI)
            _gwait(ds, k)
            _compute(ds)
            if k == 0:
                @pl.when(jj >= 1)
                def _():
                    _sdrain((k + 3) % NRING, (k + 7) % NI)
            else:
                _sdrain((k + 3) % NRING, (k + 7) % NI)
            if k >= 6:
                @pl.when(jj < (NCHUNK - 10) // NI)
                def _():
                    _iload(cc + 4, (k + 4) % NI)
            else:
                _iload(cc + 4, (k + 4) % NI)
            _scat(ds, k)
        return carry

    lax.fori_loop(0, (NCHUNK - 2) // NI, _outer, 0)
    _gwait(0, 0)
    _compute(0)
    _sdrain(3, 7)
    _scat(0, 0)
    _gwait(1, 1)
    _compute(1)
    _scat(1, 1)
    _sdrain(0, 0)
    _sdrain(1, 1)
    plsc.subcore_barrier()

    pltpu.async_copy(raw_acc.at[pl.ds(base_r, RPT)],
                     raw_out.at[c, pl.ds(base_r, RPT)], gsem[0])
    pltpu.async_copy(den_acc.at[pl.ds(base_r, RPT)],
                     den_out.at[c, pl.ds(base_r, RPT)], ssem[0])
    pltpu.make_async_copy(raw_acc.at[pl.ds(base_r, RPT)],
                          raw_out.at[c, pl.ds(base_r, RPT)], gsem[0]).wait()
    pltpu.make_async_copy(den_acc.at[pl.ds(base_r, RPT)],
                          den_out.at[c, pl.ds(base_r, RPT)], ssem[0]).wait()


def _edge_body_flat(row_hbm, col_hbm, ta_hbm, tb_hbm, xp_hbm,
                    raw_out, den_out, *rest):
    ri = rest[0:8]
    ci = rest[8:16]
    tr = rest[16:20]
    tc = rest[20:24]
    w = rest[24:28]
    xp = rest[28:32]
    raw_acc, den_acc = rest[32], rest[33]
    isem = rest[34:42]
    gsem = rest[42:46]
    ssem = rest[46:50]
    _edge_body(row_hbm, col_hbm, ta_hbm, tb_hbm, xp_hbm, raw_out, den_out,
               ri, ci, tr, tc, w, xp, raw_acc, den_acc, isem, gsem, ssem)


def _edge_pass(row, col, ta, tb, xp):
    mesh = plsc.VectorSubcoreMesh(core_axis_name="c", subcore_axis_name="s")
    f = functools.partial(
        pl.kernel,
        mesh=mesh,
        compiler_params=pltpu.CompilerParams(use_tc_tiling_on_sc=False),
        out_type=[
            jax.ShapeDtypeStruct((NC, NP, HC), jnp.float32),
            jax.ShapeDtypeStruct((NC, NP, 2 * HEADS), jnp.float32),
        ],
        scratch_types=[
            *[pltpu.VMEM((K,), jnp.int32) for _ in range(2 * NRING)],
            *[pltpu.VMEM((K,), jnp.int32) for _ in range(2 * NRING)],
            *[pltpu.VMEM((K, 2 * HEADS), jnp.float32)
              for _ in range(3 * NRING)],
            *[pltpu.VMEM((K, HC), jnp.float32) for _ in range(NRING)],
            pltpu.VMEM_SHARED((NP, HC), jnp.float32),
            pltpu.VMEM_SHARED((NP, 2 * HEADS), jnp.float32),
            *[pltpu.SemaphoreType.DMA for _ in range(4 * NRING)],
        ],
    )(_edge_body_flat)
    return f(row, col, ta, tb, xp)


def _norm_body(raw_ref, den_ref, r_ref, o_ref):
    r = raw_ref[0] + raw_ref[1]
    d = den_ref[0] + den_ref[1]
    dx = jnp.dot(d, r_ref[...], preferred_element_type=jnp.float32)
    o_ref[...] = r / (dx + 1e-16)


def _normalize(raw, den, Rmat):
    NB = 10
    R = N_NODES // NB
    return pl.pallas_call(
        _norm_body,
        grid=(NB,),
        in_specs=[
            pl.BlockSpec((NC, R, HC), lambda i: (0, i, 0)),
            pl.BlockSpec((NC, R, 2 * HEADS), lambda i: (0, i, 0)),
            pl.BlockSpec((2 * HEADS, HC), lambda i: (0, 0)),
        ],
        out_specs=pl.BlockSpec((R, HC), lambda i: (i, 0)),
        out_shape=jax.ShapeDtypeStruct((N_NODES, HC), jnp.float32),
    )(raw, den, Rmat)


def _block_diag(a):
    cc = jnp.arange(HC)
    sel = (cc[:, None] // OUT_CH) == jnp.arange(HEADS)[None, :]
    vals = a.reshape(HC)[cc]
    return sel.astype(jnp.float32) * vals[:, None]


def kernel(x, edge_index, W, a_src, a_dst):
    ei = edge_index.astype(jnp.int32)
    row, col = ei[0], ei[1]
    Ma = jnp.concatenate([_block_diag(a_src), _block_diag(a_dst)], axis=1)
    Mb = jnp.concatenate([_block_diag(a_dst), _block_diag(a_src)], axis=1)
    jj = jnp.arange(HC)[None, :]
    hh = jnp.arange(2 * HEADS)[:, None]
    Rmat = ((jj // OUT_CH == hh) & (hh < HEADS)).astype(jnp.float32)

    xp, ta, tb = _project(x, W, Ma, Mb)
    raw, den = _edge_pass(row, col, ta, tb, xp)
    return _normalize(raw, den, Rmat)
